# Initial kernel scaffold; baseline (speedup 1.0000x reference)
import jax
import jax.numpy as jnp
from jax.experimental import pallas as pl


def kernel(triples, ent_emb, rel_emb, w, b):
    raise NotImplementedError("write your pallas kernel here")


# speedup vs baseline: 2.5643x; 2.5643x over previous
import functools

import jax
import jax.numpy as jnp
from jax import lax
from jax.experimental import pallas as pl
from jax.experimental.pallas import tpu as pltpu
from jax.experimental.pallas import tpu_sc as plsc

N_TRIPLES = 160000
EMB_DIM = 256
LANES = 16
NUM_CORES = 2
NUM_SUBCORES = 16
NW = NUM_CORES * NUM_SUBCORES

CHUNK = 64
NCHUNK = N_TRIPLES // CHUNK
CPW = (NCHUNK + NW - 1) // NW
DREGS = EMB_DIM // LANES
GROUPS = CHUNK // LANES


def _sc_body(hidx_hbm, ridx_hbm, tidx_hbm, ent_hbm, rel_hbm, wb_hbm, out_hbm,
             hi, ri, ti, hrows, rrows, trows, ob, wbv, sem):
    wid = lax.axis_index("s") * NUM_CORES + lax.axis_index("c")

    pltpu.sync_copy(wb_hbm, wbv)
    wv = wbv[0, :]
    bv = wbv[1, :]

    def do_chunk(i, _):
        g = i * NW + wid

        @pl.when(g < NCHUNK)
        def _():
            base = pl.multiple_of(g * CHUNK, 8)
            pltpu.sync_copy(hidx_hbm.at[pl.ds(base, CHUNK)], hi)
            pltpu.sync_copy(ridx_hbm.at[pl.ds(base, CHUNK)], ri)
            pltpu.sync_copy(tidx_hbm.at[pl.ds(base, CHUNK)], ti)
            pltpu.async_copy(ent_hbm.at[hi], hrows, sem)
            pltpu.async_copy(rel_hbm.at[ri], rrows, sem)
            pltpu.async_copy(ent_hbm.at[ti], trows, sem)
            pltpu.make_async_copy(ent_hbm.at[hi], hrows, sem).wait()
            pltpu.make_async_copy(rel_hbm.at[ri], rrows, sem).wait()
            pltpu.make_async_copy(ent_hbm.at[ti], trows, sem).wait()

            lane_iota = lax.iota(jnp.int32, LANES)

            for grp in range(GROUPS):
                def triple_body(j, score):
                    row = grp * LANES + j
                    acc = (hrows[row, pl.ds(0, LANES)]
                           * rrows[row, pl.ds(0, LANES)]
                           * trows[row, pl.ds(0, LANES)])
                    for k in range(1, DREGS):
                        sl = pl.ds(k * LANES, LANES)
                        acc = acc + (hrows[row, sl] * rrows[row, sl]
                                     * trows[row, sl])
                    s = jnp.sum(acc)
                    return jnp.where(lane_iota == j, s, score)

                score = lax.fori_loop(
                    0, LANES, triple_body,
                    jnp.zeros((LANES,), jnp.float32), unroll=2)
                score = 1.0 / (1.0 + jnp.exp(-(wv * score + bv)))
                ob[pl.ds(grp * LANES, LANES)] = score

            pltpu.sync_copy(ob, out_hbm.at[pl.ds(base, CHUNK)])
        return 0

    lax.fori_loop(0, CPW, do_chunk, 0)


def kernel(triples, ent_emb, rel_emb, w, b):
    h_idx = triples[:, 0].astype(jnp.int32)
    r_idx = triples[:, 1].astype(jnp.int32)
    t_idx = triples[:, 2].astype(jnp.int32)
    wb = jnp.stack([
        jnp.full((LANES,), w, jnp.float32),
        jnp.full((LANES,), b, jnp.float32),
    ])

    mesh = plsc.VectorSubcoreMesh(
        core_axis_name="c", subcore_axis_name="s",
        num_cores=NUM_CORES, num_subcores=NUM_SUBCORES)

    sc_call = functools.partial(
        pl.kernel,
        mesh=mesh,
        compiler_params=pltpu.CompilerParams(needs_layout_passes=False),
        out_type=jax.ShapeDtypeStruct((N_TRIPLES,), jnp.float32),
        scratch_types=[
            pltpu.VMEM((CHUNK,), jnp.int32),
            pltpu.VMEM((CHUNK,), jnp.int32),
            pltpu.VMEM((CHUNK,), jnp.int32),
            pltpu.VMEM((CHUNK, EMB_DIM), jnp.float32),
            pltpu.VMEM((CHUNK, EMB_DIM), jnp.float32),
            pltpu.VMEM((CHUNK, EMB_DIM), jnp.float32),
            pltpu.VMEM((CHUNK,), jnp.float32),
            pltpu.VMEM((2, LANES), jnp.float32),
            pltpu.SemaphoreType.DMA,
        ],
    )(_sc_body)

    return sc_call(h_idx, r_idx, t_idx, ent_emb, rel_emb, wb)


# speedup vs baseline: 4.9029x; 1.9120x over previous
import functools

import jax
import jax.numpy as jnp
from jax import lax
from jax.experimental import pallas as pl
from jax.experimental.pallas import tpu as pltpu
from jax.experimental.pallas import tpu_sc as plsc

N_TRIPLES = 160000
EMB_DIM = 256
LANES = 16
NUM_CORES = 2
NUM_SUBCORES = 16
NW = NUM_CORES * NUM_SUBCORES
PER_W = N_TRIPLES // NW

CHUNK = 64
NCHUNK_W = 80
LAST_START = PER_W - CHUNK
DREGS = EMB_DIM // LANES
GROUPS = CHUNK // LANES


def _sc_body(hidx_hbm, ridx_hbm, tidx_hbm, ent_hbm, rel_hbm, wb_hbm, out_hbm,
             hix, rix, tix, hrows, rrows, trows, ob, wbv, sem0, sem1):
    wid = lax.axis_index("s") * NUM_CORES + lax.axis_index("c")
    base_w = pl.multiple_of(wid * PER_W, 8)

    pltpu.sync_copy(wb_hbm, wbv)
    wv = wbv[0, :]
    bv = wbv[1, :]
    pltpu.sync_copy(hidx_hbm.at[pl.ds(base_w, PER_W)], hix)
    pltpu.sync_copy(ridx_hbm.at[pl.ds(base_w, PER_W)], rix)
    pltpu.sync_copy(tidx_hbm.at[pl.ds(base_w, PER_W)], tix)

    sems = (sem0, sem1)
    lane_iota = lax.iota(jnp.int32, LANES)

    def chunk_start(c):
        return pl.multiple_of(jnp.minimum(c * CHUNK, LAST_START), 8)

    def issue(c, ph):
        start = chunk_start(c)
        idx_h = hix.at[pl.ds(start, CHUNK)]
        idx_r = rix.at[pl.ds(start, CHUNK)]
        idx_t = tix.at[pl.ds(start, CHUNK)]
        pltpu.async_copy(ent_hbm.at[idx_h], hrows.at[ph], sems[ph])
        pltpu.async_copy(rel_hbm.at[idx_r], rrows.at[ph], sems[ph])
        pltpu.async_copy(ent_hbm.at[idx_t], trows.at[ph], sems[ph])

    def wait_rows(ph):
        pltpu.make_async_copy(ent_hbm.at[hix.at[pl.ds(0, CHUNK)]],
                              hrows.at[ph], sems[ph]).wait()
        pltpu.make_async_copy(rel_hbm.at[rix.at[pl.ds(0, CHUNK)]],
                              rrows.at[ph], sems[ph]).wait()
        pltpu.make_async_copy(ent_hbm.at[tix.at[pl.ds(0, CHUNK)]],
                              trows.at[ph], sems[ph]).wait()

    def compute(c, ph):
        start = chunk_start(c)
        for grp in range(GROUPS):
            def triple_body(j, score):
                row = grp * LANES + j
                acc = (hrows[ph, row, pl.ds(0, LANES)]
                       * rrows[ph, row, pl.ds(0, LANES)]
                       * trows[ph, row, pl.ds(0, LANES)])
                for k in range(1, DREGS):
                    sl = pl.ds(k * LANES, LANES)
                    acc = acc + (hrows[ph, row, sl] * rrows[ph, row, sl]
                                 * trows[ph, row, sl])
                s = jnp.sum(acc)
                return jnp.where(lane_iota == j, s, score)

            score = lax.fori_loop(
                0, LANES, triple_body,
                jnp.zeros((LANES,), jnp.float32), unroll=2)
            score = 1.0 / (1.0 + jnp.exp(-(wv * score + bv)))
            ob[pl.ds(start + grp * LANES, LANES)] = score

    issue(0, 0)

    def pair_body(i2, _):
        a = 2 * i2
        issue(a + 1, 1)
        wait_rows(0)
        compute(a, 0)

        @pl.when(a + 2 < NCHUNK_W)
        def _():
            issue(a + 2, 0)

        wait_rows(1)
        compute(a + 1, 1)
        return 0

    lax.fori_loop(0, NCHUNK_W // 2, pair_body, 0)

    pltpu.sync_copy(ob, out_hbm.at[pl.ds(base_w, PER_W)])


def kernel(triples, ent_emb, rel_emb, w, b):
    h_idx = triples[:, 0].astype(jnp.int32)
    r_idx = triples[:, 1].astype(jnp.int32)
    t_idx = triples[:, 2].astype(jnp.int32)
    wb = jnp.stack([
        jnp.full((LANES,), w, jnp.float32),
        jnp.full((LANES,), b, jnp.float32),
    ])

    mesh = plsc.VectorSubcoreMesh(
        core_axis_name="c", subcore_axis_name="s",
        num_cores=NUM_CORES, num_subcores=NUM_SUBCORES)

    sc_call = functools.partial(
        pl.kernel,
        mesh=mesh,
        compiler_params=pltpu.CompilerParams(needs_layout_passes=False),
        out_type=jax.ShapeDtypeStruct((N_TRIPLES,), jnp.float32),
        scratch_types=[
            pltpu.VMEM((PER_W,), jnp.int32),
            pltpu.VMEM((PER_W,), jnp.int32),
            pltpu.VMEM((PER_W,), jnp.int32),
            pltpu.VMEM((2, CHUNK, EMB_DIM), jnp.float32),
            pltpu.VMEM((2, CHUNK, EMB_DIM), jnp.float32),
            pltpu.VMEM((2, CHUNK, EMB_DIM), jnp.float32),
            pltpu.VMEM((PER_W,), jnp.float32),
            pltpu.VMEM((2, LANES), jnp.float32),
            pltpu.SemaphoreType.DMA,
            pltpu.SemaphoreType.DMA,
        ],
    )(_sc_body)

    return sc_call(h_idx, r_idx, t_idx, ent_emb, rel_emb, wb)
